# baseline (speedup 1.0000x reference)
import functools

import jax
import jax.numpy as jnp
from jax import lax
from jax.experimental import pallas as pl
from jax.experimental.pallas import tpu as pltpu
from jax.experimental.pallas import tpu_sc as plsc

DIM = 128
CHUNK = 128

try:
    _info = plsc.get_sparse_core_info()
    NC, NS = _info.num_cores, _info.num_subcores
except Exception:
    NC, NS = 2, 16
NW = NC * NS


def _sc_gather(table, ids3):
    nw, nchunk, chunk = ids3.shape
    n = nw * nchunk * chunk
    per_w = n // NW
    mesh = plsc.VectorSubcoreMesh(core_axis_name="c", subcore_axis_name="s")

    @functools.partial(
        pl.kernel,
        out_type=jax.ShapeDtypeStruct((n, DIM), jnp.float32),
        mesh=mesh,
        scratch_types=[
            pltpu.VMEM((nchunk, CHUNK), jnp.int32),
            pltpu.VMEM((CHUNK, DIM), jnp.float32),
            pltpu.VMEM((CHUNK, DIM), jnp.float32),
            pltpu.SemaphoreType.DMA,
            pltpu.SemaphoreType.DMA,
        ],
    )
    def gather_kernel(table_hbm, ids_hbm, out_hbm, idx_v, buf0, buf1, sem0, sem1):
        wid = lax.axis_index("s") * NC + lax.axis_index("c")
        base = wid * per_w
        pltpu.sync_copy(ids_hbm.at[wid], idx_v)
        bufs = (buf0, buf1)
        sems = (sem0, sem1)

        def start(j, k):
            pltpu.make_async_copy(
                table_hbm.at[idx_v.at[j]], bufs[k], sems[k]
            ).start()

        def finish(j, k):
            pltpu.make_async_copy(
                table_hbm.at[idx_v.at[j]], bufs[k], sems[k]
            ).wait()
            pltpu.sync_copy(bufs[k], out_hbm.at[pl.ds(base + j * CHUNK, CHUNK)])

        start(0, 0)
        start(1, 1)

        def body(i, carry):
            j = 2 * i
            finish(j, 0)

            @pl.when(j + 2 < nchunk)
            def _():
                start(j + 2, 0)

            finish(j + 1, 1)

            @pl.when(j + 3 < nchunk)
            def _():
                start(j + 3, 1)

            return carry

        lax.fori_loop(0, nchunk // 2, body, 0)

    return gather_kernel(table, ids3)


L_BLK = 2
K = 5


def _mm_body(x_ref, w_ref, b_ref, o_ref):
    y = (
        lax.dot_general(
            x_ref[...].astype(jnp.bfloat16),
            w_ref[...].astype(jnp.bfloat16),
            (((1,), (1,)), ((), ())),
            preferred_element_type=jnp.float32,
        )
        + b_ref[...]
    )
    o_ref[...] = y.reshape(L_BLK, x_ref.shape[0] // L_BLK, DIM)


def _tc_project_chunk(g, W, b, out_prev, B, L, l_off):
    lk = g.shape[0] // B
    grid = (lk // L_BLK,)
    x_spec = pl.BlockSpec((L_BLK * B, DIM), lambda i: (i, 0))
    w_spec = pl.BlockSpec((DIM, DIM), lambda i: (0, 0))
    b_spec = pl.BlockSpec((DIM,), lambda i: (0,))
    blk = l_off // L_BLK
    o_spec = pl.BlockSpec((L_BLK, B, DIM), lambda i: (blk + i, 0, 0))
    out_shape = jax.ShapeDtypeStruct((L, B, DIM), jnp.float32)

    def mm_first(x_ref, w_ref, b_ref, o_ref):
        _mm_body(x_ref, w_ref, b_ref, o_ref)

    if out_prev is None:
        return pl.pallas_call(
            mm_first,
            grid=grid,
            in_specs=[x_spec, w_spec, b_spec],
            out_specs=o_spec,
            out_shape=out_shape,
        )(g, W, b)

    def mm_alias(prev_ref, x_ref, w_ref, b_ref, o_ref):
        _mm_body(x_ref, w_ref, b_ref, o_ref)

    return pl.pallas_call(
        mm_alias,
        grid=grid,
        in_specs=[
            pl.BlockSpec(memory_space=pltpu.MemorySpace.HBM),
            x_spec,
            w_spec,
            b_spec,
        ],
        out_specs=o_spec,
        out_shape=out_shape,
        input_output_aliases={0: 0},
    )(out_prev, g, W, b)


def kernel(news_ids, news_categ, table, W, b):
    B, L = news_ids.shape
    n = B * L
    lk = L // K
    ids4 = jnp.transpose(news_ids).reshape(K, NW, n // (K * NW * CHUNK), CHUNK)
    ids4 = ids4.astype(jnp.int32)
    gs = [_sc_gather(table, ids4[k]) for k in range(K)]
    out = None
    for k in range(K):
        out = _tc_project_chunk(gs[k], W, b, out, B, L, k * lk)
    return jnp.transpose(out, (1, 0, 2))


# speedup vs baseline: 1.1412x; 1.1412x over previous
import functools

import jax
import jax.numpy as jnp
from jax import lax
from jax.experimental import pallas as pl
from jax.experimental.pallas import tpu as pltpu
from jax.experimental.pallas import tpu_sc as plsc

DIM = 128
CHUNK = 128

try:
    _info = plsc.get_sparse_core_info()
    NC, NS = _info.num_cores, _info.num_subcores
except Exception:
    NC, NS = 2, 16
NW = NC * NS


def _sc_gather_pack(table, ids3):
    nw, nchunk, chunk = ids3.shape
    n = nw * nchunk * chunk
    per_w = n // NW
    half = chunk // 2
    mesh = plsc.VectorSubcoreMesh(core_axis_name="c", subcore_axis_name="s")

    @functools.partial(
        pl.kernel,
        out_type=jax.ShapeDtypeStruct((n // 2, DIM), jnp.int32),
        mesh=mesh,
        scratch_types=[
            pltpu.VMEM((nchunk, CHUNK), jnp.int32),
            pltpu.VMEM((CHUNK, DIM), jnp.float32),
            pltpu.VMEM((CHUNK, DIM), jnp.float32),
            pltpu.VMEM((CHUNK // 2, DIM), jnp.int32),
            pltpu.VMEM((CHUNK // 2, DIM), jnp.int32),
            pltpu.SemaphoreType.DMA,
            pltpu.SemaphoreType.DMA,
        ],
    )
    def gather_kernel(
        table_hbm, ids_hbm, out_hbm, idx_v, buf0, buf1, bufi0, bufi1, sem0, sem1
    ):
        wid = lax.axis_index("s") * NC + lax.axis_index("c")
        base2 = wid * (per_w // 2)
        pltpu.sync_copy(ids_hbm.at[wid], idx_v)
        bufs = (buf0, buf1)
        bufis = (bufi0, bufi1)
        sems = (sem0, sem1)

        def start(j, k):
            pltpu.make_async_copy(
                table_hbm.at[idx_v.at[j]], bufs[k], sems[k]
            ).start()

        def finish(j, k):
            pltpu.make_async_copy(
                table_hbm.at[idx_v.at[j]], bufs[k], sems[k]
            ).wait()
            buf, bufi = bufs[k], bufis[k]

            def pack_row(t, carry):
                for cs in range(DIM // 16):
                    a = buf[t, pl.ds(cs * 16, 16)]
                    c = buf[t + half, pl.ds(cs * 16, 16)]
                    za = lax.shift_right_logical(
                        lax.bitcast_convert_type(a, jnp.int32)
                        + jnp.int32(0x8000),
                        16,
                    )
                    zc = (
                        lax.bitcast_convert_type(c, jnp.int32)
                        + jnp.int32(0x8000)
                    ) & jnp.int32(-65536)
                    bufi[t, pl.ds(cs * 16, 16)] = za | zc
                return carry

            lax.fori_loop(0, half, pack_row, 0)
            pltpu.sync_copy(bufi, out_hbm.at[pl.ds(base2 + j * half, half)])

        start(0, 0)
        start(1, 1)

        def body(i, carry):
            j = 2 * i
            finish(j, 0)

            @pl.when(j + 2 < nchunk)
            def _():
                start(j + 2, 0)

            finish(j + 1, 1)

            @pl.when(j + 3 < nchunk)
            def _():
                start(j + 3, 1)

            return carry

        lax.fori_loop(0, nchunk // 2, body, 0)

    return gather_kernel(table, ids3)


L_BLK = 2
K = 5


def _mm_body(x_ref, w_ref, b_ref, o_ref, B):
    xi = x_ref[...]
    xlo = lax.bitcast_convert_type(
        lax.shift_left(xi, 16), jnp.float32
    ).astype(jnp.bfloat16)
    xhi = lax.bitcast_convert_type(
        xi & jnp.int32(-65536), jnp.float32
    ).astype(jnp.bfloat16)
    dn = (((1,), (1,)), ((), ()))
    wb = w_ref[...].astype(jnp.bfloat16)
    ylo = lax.dot_general(xlo, wb, dn, preferred_element_type=jnp.float32)
    yhi = lax.dot_general(xhi, wb, dn, preferred_element_type=jnp.float32)
    r2 = xi.shape[0]
    nch = r2 // 64
    y = jnp.concatenate(
        [ylo.reshape(nch, 1, 64, DIM), yhi.reshape(nch, 1, 64, DIM)],
        axis=1,
    ).reshape(2 * r2, DIM) + b_ref[...]
    o_ref[...] = y.reshape(L_BLK, B, DIM)


def _tc_project_chunk(gp, W, b, out_prev, B, L, l_off):
    lk = 2 * gp.shape[0] // B
    grid = (lk // L_BLK,)
    r2 = L_BLK * B // 2
    x_spec = pl.BlockSpec((r2, DIM), lambda i: (i, 0))
    w_spec = pl.BlockSpec((DIM, DIM), lambda i: (0, 0))
    b_spec = pl.BlockSpec((DIM,), lambda i: (0,))
    blk = l_off // L_BLK
    o_spec = pl.BlockSpec((L_BLK, B, DIM), lambda i: (blk + i, 0, 0))
    out_shape = jax.ShapeDtypeStruct((L, B, DIM), jnp.float32)

    def mm_first(x_ref, w_ref, b_ref, o_ref):
        _mm_body(x_ref, w_ref, b_ref, o_ref, B)

    if out_prev is None:
        return pl.pallas_call(
            mm_first,
            grid=grid,
            in_specs=[x_spec, w_spec, b_spec],
            out_specs=o_spec,
            out_shape=out_shape,
        )(gp, W, b)

    def mm_alias(prev_ref, x_ref, w_ref, b_ref, o_ref):
        _mm_body(x_ref, w_ref, b_ref, o_ref, B)

    return pl.pallas_call(
        mm_alias,
        grid=grid,
        in_specs=[
            pl.BlockSpec(memory_space=pltpu.MemorySpace.HBM),
            x_spec,
            w_spec,
            b_spec,
        ],
        out_specs=o_spec,
        out_shape=out_shape,
        input_output_aliases={0: 0},
    )(out_prev, gp, W, b)


def kernel(news_ids, news_categ, table, W, b):
    B, L = news_ids.shape
    n = B * L
    lk = L // K
    ids4 = jnp.transpose(news_ids).reshape(K, NW, n // (K * NW * CHUNK), CHUNK)
    ids4 = ids4.astype(jnp.int32)
    gps = [_sc_gather_pack(table, ids4[k]) for k in range(K)]
    out = None
    for k in range(K):
        out = _tc_project_chunk(gps[k], W, b, out, B, L, k * lk)
    return jnp.transpose(out, (1, 0, 2))


# speedup vs baseline: 1.1704x; 1.0256x over previous
import functools

import jax
import jax.numpy as jnp
from jax import lax
from jax.experimental import pallas as pl
from jax.experimental.pallas import tpu as pltpu
from jax.experimental.pallas import tpu_sc as plsc

DIM = 128
CHUNK = 128

try:
    _info = plsc.get_sparse_core_info()
    NC, NS = _info.num_cores, _info.num_subcores
except Exception:
    NC, NS = 2, 16
NW = NC * NS


def _sc_gather_pack(table, ids3):
    nw, nchunk, chunk = ids3.shape
    n = nw * nchunk * chunk
    per_w = n // NW
    half = chunk // 2
    mesh = plsc.VectorSubcoreMesh(core_axis_name="c", subcore_axis_name="s")

    @functools.partial(
        pl.kernel,
        out_type=jax.ShapeDtypeStruct((n // 2, DIM), jnp.int32),
        mesh=mesh,
        scratch_types=[
            pltpu.VMEM((nchunk, CHUNK), jnp.int32),
            pltpu.VMEM((CHUNK, DIM), jnp.float32),
            pltpu.VMEM((CHUNK, DIM), jnp.float32),
            pltpu.VMEM((CHUNK // 2, DIM), jnp.int32),
            pltpu.VMEM((CHUNK // 2, DIM), jnp.int32),
            pltpu.SemaphoreType.DMA,
            pltpu.SemaphoreType.DMA,
            pltpu.SemaphoreType.DMA,
            pltpu.SemaphoreType.DMA,
        ],
    )
    def gather_kernel(
        table_hbm,
        ids_hbm,
        out_hbm,
        idx_v,
        buf0,
        buf1,
        bufi0,
        bufi1,
        sem0,
        sem1,
        ssem0,
        ssem1,
    ):
        wid = lax.axis_index("s") * NC + lax.axis_index("c")
        base2 = wid * (per_w // 2)
        pltpu.sync_copy(ids_hbm.at[wid], idx_v)
        bufs = (buf0, buf1)
        bufis = (bufi0, bufi1)
        sems = (sem0, sem1)
        ssems = (ssem0, ssem1)

        def start(j, k):
            pltpu.make_async_copy(
                table_hbm.at[idx_v.at[j]], bufs[k], sems[k]
            ).start()

        def store_copy(j, k):
            return pltpu.make_async_copy(
                bufis[k], out_hbm.at[pl.ds(base2 + j * half, half)], ssems[k]
            )

        def finish(j, k, drain):
            pltpu.make_async_copy(
                table_hbm.at[idx_v.at[j]], bufs[k], sems[k]
            ).wait()
            if drain:
                store_copy(j - 2, k).wait()
            buf, bufi = bufs[k], bufis[k]

            def pack_row(t, carry):
                for cs in range(DIM // 16):
                    a = buf[t, pl.ds(cs * 16, 16)]
                    c = buf[t + half, pl.ds(cs * 16, 16)]
                    za = lax.shift_right_logical(
                        lax.bitcast_convert_type(a, jnp.int32)
                        + jnp.int32(0x8000),
                        16,
                    )
                    zc = (
                        lax.bitcast_convert_type(c, jnp.int32)
                        + jnp.int32(0x8000)
                    ) & jnp.int32(-65536)
                    bufi[t, pl.ds(cs * 16, 16)] = za | zc
                return carry

            lax.fori_loop(0, half, pack_row, 0)
            store_copy(j, k).start()

        start(0, 0)
        start(1, 1)
        finish(0, 0, drain=False)
        start(2, 0)
        finish(1, 1, drain=False)
        start(3, 1)

        def body(i, carry):
            j = 2 * i
            finish(j, 0, drain=True)

            @pl.when(j + 2 < nchunk)
            def _():
                start(j + 2, 0)

            finish(j + 1, 1, drain=True)

            @pl.when(j + 3 < nchunk)
            def _():
                start(j + 3, 1)

            return carry

        lax.fori_loop(1, nchunk // 2, body, 0)
        store_copy(nchunk - 2, 0).wait()
        store_copy(nchunk - 1, 1).wait()

    return gather_kernel(table, ids3)


L_BLK = 2
K = 5


def _mm_body(x_ref, w_ref, b_ref, o_ref, B):
    xi = x_ref[...]
    xlo = lax.bitcast_convert_type(
        lax.shift_left(xi, 16), jnp.float32
    ).astype(jnp.bfloat16)
    xhi = lax.bitcast_convert_type(
        xi & jnp.int32(-65536), jnp.float32
    ).astype(jnp.bfloat16)
    dn = (((1,), (1,)), ((), ()))
    wb = w_ref[...].astype(jnp.bfloat16)
    ylo = lax.dot_general(xlo, wb, dn, preferred_element_type=jnp.float32)
    yhi = lax.dot_general(xhi, wb, dn, preferred_element_type=jnp.float32)
    r2 = xi.shape[0]
    nch = r2 // 64
    y = jnp.concatenate(
        [ylo.reshape(nch, 1, 64, DIM), yhi.reshape(nch, 1, 64, DIM)],
        axis=1,
    ).reshape(2 * r2, DIM) + b_ref[...]
    o_ref[...] = y.reshape(L_BLK, B, DIM)


def _tc_project_chunk(gp, W, b, out_prev, B, L, l_off):
    lk = 2 * gp.shape[0] // B
    grid = (lk // L_BLK,)
    r2 = L_BLK * B // 2
    x_spec = pl.BlockSpec((r2, DIM), lambda i: (i, 0))
    w_spec = pl.BlockSpec((DIM, DIM), lambda i: (0, 0))
    b_spec = pl.BlockSpec((DIM,), lambda i: (0,))
    blk = l_off // L_BLK
    o_spec = pl.BlockSpec((L_BLK, B, DIM), lambda i: (blk + i, 0, 0))
    out_shape = jax.ShapeDtypeStruct((L, B, DIM), jnp.float32)

    def mm_first(x_ref, w_ref, b_ref, o_ref):
        _mm_body(x_ref, w_ref, b_ref, o_ref, B)

    if out_prev is None:
        return pl.pallas_call(
            mm_first,
            grid=grid,
            in_specs=[x_spec, w_spec, b_spec],
            out_specs=o_spec,
            out_shape=out_shape,
        )(gp, W, b)

    def mm_alias(prev_ref, x_ref, w_ref, b_ref, o_ref):
        _mm_body(x_ref, w_ref, b_ref, o_ref, B)

    return pl.pallas_call(
        mm_alias,
        grid=grid,
        in_specs=[
            pl.BlockSpec(memory_space=pltpu.MemorySpace.HBM),
            x_spec,
            w_spec,
            b_spec,
        ],
        out_specs=o_spec,
        out_shape=out_shape,
        input_output_aliases={0: 0},
    )(out_prev, gp, W, b)


def kernel(news_ids, news_categ, table, W, b):
    B, L = news_ids.shape
    n = B * L
    lk = L // K
    ids4 = jnp.transpose(news_ids).reshape(K, NW, n // (K * NW * CHUNK), CHUNK)
    ids4 = ids4.astype(jnp.int32)
    gps = [_sc_gather_pack(table, ids4[k]) for k in range(K)]
    out = None
    for k in range(K):
        out = _tc_project_chunk(gps[k], W, b, out, B, L, k * lk)
    return jnp.transpose(out, (1, 0, 2))
